# baseline (speedup 1.0000x reference)
import math

import jax
import jax.numpy as jnp
import numpy as np
from jax.experimental import pallas as pl
from jax.experimental.pallas import tpu as pltpu

BATCH = 4096
HIST = 200
D_MODEL = 128
NUM_INDICES = BATCH * HIST
ROWS = NUM_INDICES // 128
BS = 8


def _div_phase():
    j = np.arange(0, D_MODEL, 2, dtype=np.float32)
    div = np.exp(j * (-math.log(10000.0) / D_MODEL)).astype(np.float32)
    div_full = np.repeat(div, 2)[None, :]
    phase = np.tile(np.array([0.0, np.pi / 2], dtype=np.float32), D_MODEL // 2)[None, :]
    return jnp.asarray(div_full), jnp.asarray(phase)


def _tc_body(ts_ref, div_ref, ph_ref, o_ref):
    t = ts_ref[...].astype(jnp.float32)
    tt = t.T
    div = div_ref[...]
    ph = ph_ref[...]
    for j in range(BS):
        col = tt[:, j : j + 1]
        args = col * div + ph
        o_ref[j] = jnp.sin(args)


def kernel(timesteps, pe):
    del pe
    ts2d = timesteps.reshape(ROWS, 128)
    div_full, phase = _div_phase()

    out = pl.pallas_call(
        _tc_body,
        grid=(ROWS // BS,),
        in_specs=[
            pl.BlockSpec((BS, 128), lambda i: (i, 0)),
            pl.BlockSpec((1, D_MODEL), lambda i: (0, 0)),
            pl.BlockSpec((1, D_MODEL), lambda i: (0, 0)),
        ],
        out_specs=pl.BlockSpec((BS, 128, D_MODEL), lambda i: (i, 0, 0)),
        out_shape=jax.ShapeDtypeStruct((ROWS, 128, D_MODEL), jnp.float32),
    )(ts2d, div_full, phase)
    return out.reshape(BATCH, HIST, D_MODEL)


# speedup vs baseline: 4.5969x; 4.5969x over previous
import jax
import jax.numpy as jnp
from jax.experimental import pallas as pl
from jax.experimental.pallas import tpu as pltpu
from jax.experimental.pallas import tpu_sc as plsc

BATCH = 4096
HIST = 200
D_MODEL = 128
NUM_INDICES = BATCH * HIST
WINDOW = 256


def kernel(timesteps, pe):
    indices = timesteps.reshape((1, NUM_INDICES))

    vector_mesh = plsc.VectorSubcoreMesh(
        core_axis_name="core", subcore_axis_name="subcore"
    )

    @jax.jit
    def gather(pe, indices):
        @pl.kernel(
            out_type=jax.ShapeDtypeStruct((NUM_INDICES, D_MODEL), pe.dtype),
            mesh=vector_mesh,
        )
        def sc_kernel(pe_hbm, i_hbm, o_hbm):
            def body(i_vmem, o_vmem):
                pltpu.sync_copy(pe_hbm.at[i_vmem.at[0]], o_vmem)

            pltpu.emit_pipeline(
                body,
                grid=(NUM_INDICES // WINDOW,),
                in_specs=[
                    pl.BlockSpec((1, WINDOW), index_map=lambda i: (0, i))
                ],
                out_specs=[
                    pl.BlockSpec((WINDOW, D_MODEL), index_map=lambda i: (i, 0))
                ],
                core_axis_name=("core", "subcore"),
                dimension_semantics=(pltpu.PARALLEL,),
            )(i_hbm, o_hbm)

        return sc_kernel(pe, indices)

    out = gather(pe, indices)
    return out.reshape((BATCH, HIST, D_MODEL))
